# Initial kernel scaffold; baseline (speedup 1.0000x reference)
import jax
import jax.numpy as jnp
from jax.experimental import pallas as pl


def kernel(x, qweights, fc1_w, fc1_b, fc2_w, fc2_b):
    raise NotImplementedError("write your pallas kernel here")


# speedup vs baseline: 26.7366x; 26.7366x over previous
import numpy as np
import jax
import jax.numpy as jnp
from jax.experimental import pallas as pl
from jax.experimental.pallas import tpu as pltpu

N_Q = 4
N_KER = 4
K, STRIDE, HOUT = 4, 2, 3
EPS = 1e-12
NPATCH = HOUT * HOUT
NFEAT = 3 * N_KER
BBLK = 2048


def _patch_matrix():
    G = np.zeros((64, NPATCH * 16), np.float32)
    for i in range(HOUT):
        for j in range(HOUT):
            q = i * HOUT + j
            for r in range(K):
                for c in range(K):
                    G[8 * (STRIDE * i + r) + STRIDE * j + c, q * 16 + r * K + c] = 1.0
    return G


def _sel_matrix():
    S = np.zeros((NFEAT * 16, NFEAT), np.float32)
    for j in range(NFEAT):
        S[j * 16:(j + 1) * 16, j] = 1.0
    return S


def _perm():
    idx = np.zeros(NPATCH * NFEAT, np.int32)
    for q in range(NPATCH):
        for j in range(NFEAT):
            idx[q * NFEAT + j] = j * NPATCH + q
    return idx


def _cnot_ring():
    P = np.eye(16, dtype=np.float32)

    def cnot(c_, t_):
        M = np.zeros((16, 16), np.float32)
        for n in range(16):
            bits = [(n >> (3 - w)) & 1 for w in range(N_Q)]
            if bits[c_]:
                bits[t_] ^= 1
            m = sum(b << (3 - w) for w, b in enumerate(bits))
            M[m, n] = 1.0
        return M

    for (c_, t_) in [(0, 1), (1, 2), (2, 3), (3, 0)]:
        P = cnot(c_, t_) @ P
    return P


def _build_A(qweights):
    phi, theta, omega = qweights[..., 0], qweights[..., 1], qweights[..., 2]
    c, s = jnp.cos(theta / 2), jnp.sin(theta / 2)
    ep = jnp.exp(-0.5j * (phi + omega).astype(jnp.complex64))
    em = jnp.exp(0.5j * (phi - omega).astype(jnp.complex64))
    m00, m01, m10, m11 = ep * c, -em * s, jnp.conj(em) * s, jnp.conj(ep) * c
    R = jnp.stack([jnp.stack([m00, m01], -1),
                   jnp.stack([m10, m11], -1)], -2)

    P = jnp.asarray(_cnot_ring()).astype(jnp.complex64)
    X = np.array([[0, 1], [1, 0]], np.complex64)
    Y = np.array([[0, -1j], [1j, 0]], np.complex64)
    Z = np.array([[1, 0], [0, -1]], np.complex64)
    I8 = np.eye(8, dtype=np.complex64)
    obs = [jnp.asarray(np.kron(o, I8)) for o in (X, Y, Z)]

    hp = jax.lax.Precision.HIGHEST
    cols = []
    for k in range(N_KER):
        U = R[k, 0]
        for q in range(1, N_Q):
            U = jnp.kron(U, R[k, q])
        U = jnp.matmul(P, U, precision=hp)
        Uh = jnp.conj(U.T)
        for O in obs:
            M = jnp.matmul(Uh, jnp.matmul(O, U, precision=hp), precision=hp)
            cols.append(jnp.real(M))
    return jnp.concatenate(cols, axis=1).astype(jnp.float32)


def _body(x_ref, g_ref, a_ref, s_ref, w1_ref, b1_ref, w2_ref, b2_ref, o_ref):
    hp = jax.lax.Precision.HIGHEST
    xv = x_ref[...]
    p = jnp.dot(xv, g_ref[...], precision=hp, preferred_element_type=jnp.float32)
    A = a_ref[...]
    S = s_ref[...]
    chunks = []
    for q in range(NPATCH):
        pq = p[:, q * 16:(q + 1) * 16]
        nsq = jnp.sum(pq * pq, axis=1, keepdims=True)
        inv = 1.0 / jnp.square(jnp.sqrt(nsq) + EPS)
        T = jnp.dot(pq, A, precision=hp, preferred_element_type=jnp.float32)
        pt = jnp.concatenate([pq] * NFEAT, axis=1)
        E = jnp.dot(T * pt, S, precision=hp, preferred_element_type=jnp.float32) * inv
        chunks.append(jnp.where(E >= 0, E, 0.1 * E))
    h = jnp.concatenate(chunks, axis=1)
    h1 = jnp.dot(h, w1_ref[...], precision=hp, preferred_element_type=jnp.float32) + b1_ref[...]
    h1 = jnp.where(h1 >= 0, h1, 0.1 * h1)
    o_ref[...] = jnp.dot(h1, w2_ref[...], precision=hp,
                         preferred_element_type=jnp.float32) + b2_ref[...]


def kernel(x, qweights, fc1_w, fc1_b, fc2_w, fc2_b):
    Bsz = x.shape[0]
    xf = x.reshape(Bsz, 64)
    G = jnp.asarray(_patch_matrix())
    S = jnp.asarray(_sel_matrix())
    A = _build_A(qweights)
    W1 = fc1_w[:, jnp.asarray(_perm())].T
    W2 = fc2_w.T
    b1 = fc1_b.reshape(1, 32)
    b2 = fc2_b.reshape(1, 3)

    def const(shape):
        return pl.BlockSpec(shape, lambda i: (0, 0))

    return pl.pallas_call(
        _body,
        grid=(Bsz // BBLK,),
        in_specs=[
            pl.BlockSpec((BBLK, 64), lambda i: (i, 0)),
            const((64, NPATCH * 16)),
            const((16, NFEAT * 16)),
            const((NFEAT * 16, NFEAT)),
            const((NPATCH * NFEAT, 32)),
            const((1, 32)),
            const((32, 3)),
            const((1, 3)),
        ],
        out_specs=pl.BlockSpec((BBLK, 3), lambda i: (i, 0)),
        out_shape=jax.ShapeDtypeStruct((Bsz, 3), jnp.float32),
        compiler_params=pltpu.CompilerParams(
            dimension_semantics=("parallel",),
        ),
        name="qccnn_fused",
    )(xf, G, A, S, W1, b1, W2, b2)
